# Initial kernel scaffold; baseline (speedup 1.0000x reference)
import functools

import jax
import jax.numpy as jnp
from jax import lax
from jax.experimental import pallas as pl
from jax.experimental.pallas import tpu as pltpu
from jax.experimental.pallas import tpu_sc as plsc

L = 16
NW = 32
B = 256


def _make_sc_call(N, C, D, H, W):
    PW = N // NW
    NB = PW // B
    NG = B // L

    mesh = plsc.VectorSubcoreMesh(core_axis_name="c", subcore_axis_name="s")

    @functools.partial(
        pl.kernel,
        mesh=mesh,
        out_type=jax.ShapeDtypeStruct((N, C), jnp.float32),
        scratch_types=[
            pltpu.VMEM((6 * L,), jnp.float32),
            pltpu.VMEM((B,), jnp.float32),
            pltpu.VMEM((B,), jnp.float32),
            pltpu.VMEM((B,), jnp.float32),
            pltpu.VMEM((8 * B,), jnp.int32),
            pltpu.VMEM((8 * B,), jnp.float32),
            pltpu.VMEM((8 * B, L), jnp.float32),
            pltpu.VMEM((B, C), jnp.float32),
            pltpu.SemaphoreType.DMA,
        ],
    )
    def launch(xs_hbm, ys_hbm, zs_hbm, table_hbm, params_hbm, out_hbm,
               par_v, x_v, y_v, z_v, idx_v, w_v, rows_v, out_v, sem):
        wid = lax.axis_index("s") * 2 + lax.axis_index("c")
        base = wid * PW
        pltpu.sync_copy(params_hbm, par_v)
        iota = lax.iota(jnp.int32, L)

        def axis_vals(p, a, S):
            mn = par_v[pl.ds(a * L, L)]
            dn = par_v[pl.ds((3 + a) * L, L)]
            n = (p - mn) / dn
            cc = n * 2.0 - 1.0
            u = (cc + 1.0) * 0.5 * float(S - 1)
            u = jnp.clip(u, 0.0, float(S - 1))
            i0 = u.astype(jnp.int32)
            t = u - i0.astype(jnp.float32)
            i1 = jnp.minimum(i0 + 1, S - 1)
            return i0, i1, t

        def block_body(bi, carry):
            pbase = base + bi * B
            pltpu.sync_copy(xs_hbm.at[pl.ds(pbase, B)], x_v)
            pltpu.sync_copy(ys_hbm.at[pl.ds(pbase, B)], y_v)
            pltpu.sync_copy(zs_hbm.at[pl.ds(pbase, B)], z_v)

            def grp_a(g, c2):
                off = g * L
                iz0, iz1, tz = axis_vals(x_v[pl.ds(off, L)], 0, D)
                iy0, iy1, ty = axis_vals(y_v[pl.ds(off, L)], 1, H)
                ix0, ix1, tx = axis_vals(z_v[pl.ds(off, L)], 2, W)
                a0 = iz0 * (H * W)
                a1 = iz1 * (H * W)
                b0 = iy0 * W
                b1 = iy1 * W
                r00 = a0 + b0
                r01 = a0 + b1
                r10 = a1 + b0
                r11 = a1 + b1
                idx_v[pl.ds(0 * B + off, L)] = r00 + ix0
                idx_v[pl.ds(1 * B + off, L)] = r00 + ix1
                idx_v[pl.ds(2 * B + off, L)] = r01 + ix0
                idx_v[pl.ds(3 * B + off, L)] = r01 + ix1
                idx_v[pl.ds(4 * B + off, L)] = r10 + ix0
                idx_v[pl.ds(5 * B + off, L)] = r10 + ix1
                idx_v[pl.ds(6 * B + off, L)] = r11 + ix0
                idx_v[pl.ds(7 * B + off, L)] = r11 + ix1
                cz = 1.0 - tz
                cy = 1.0 - ty
                cx = 1.0 - tx
                w00 = cz * cy
                w01 = cz * ty
                w10 = tz * cy
                w11 = tz * ty
                w_v[pl.ds(0 * B + off, L)] = w00 * cx
                w_v[pl.ds(1 * B + off, L)] = w00 * tx
                w_v[pl.ds(2 * B + off, L)] = w01 * cx
                w_v[pl.ds(3 * B + off, L)] = w01 * tx
                w_v[pl.ds(4 * B + off, L)] = w10 * cx
                w_v[pl.ds(5 * B + off, L)] = w10 * tx
                w_v[pl.ds(6 * B + off, L)] = w11 * cx
                w_v[pl.ds(7 * B + off, L)] = w11 * tx
                return c2

            lax.fori_loop(0, NG, grp_a, 0)

            pltpu.async_copy(table_hbm.at[idx_v], rows_v, sem).wait()

            def grp_b(g, c2):
                off = g * L
                pv = off + iota
                ws = [w_v[pl.ds(k * B + off, L)] for k in range(8)]
                rvs = [pv + (k * B) for k in range(8)]
                for ch in range(C):
                    cvec = jnp.full((L,), ch, jnp.int32)
                    acc = ws[0] * plsc.load_gather(rows_v, [rvs[0], cvec])
                    for k in range(1, 8):
                        acc = acc + ws[k] * plsc.load_gather(rows_v, [rvs[k], cvec])
                    plsc.store_scatter(out_v, [pv, cvec], acc)
                return c2

            lax.fori_loop(0, NG, grp_b, 0)

            pltpu.sync_copy(out_v, out_hbm.at[pl.ds(pbase, B)])
            return carry

        lax.fori_loop(0, NB, block_body, 0)

    return launch


def kernel(xyz, grid, xyz_min, xyz_max):
    N = xyz.shape[0]
    C = grid.shape[1]
    D, H, W = grid.shape[2], grid.shape[3], grid.shape[4]
    V = D * H * W
    table = jnp.pad(grid[0].reshape(C, V).T, ((0, 0), (0, L - C)))
    xs = xyz[:, 0]
    ys = xyz[:, 1]
    zs = xyz[:, 2]
    den = xyz_max - xyz_min
    params = jnp.concatenate([
        jnp.broadcast_to(xyz_min[:, None], (3, L)).reshape(-1),
        jnp.broadcast_to(den[:, None], (3, L)).reshape(-1),
    ])
    call = _make_sc_call(N, C, D, H, W)
    return call(xs, ys, zs, table, params)


# speedup vs baseline: 1.3085x; 1.3085x over previous
import functools

import jax
import jax.numpy as jnp
from jax import lax
from jax.experimental import pallas as pl
from jax.experimental.pallas import tpu as pltpu
from jax.experimental.pallas import tpu_sc as plsc

L = 16
NW = 32
B = 128
NC = 8


def _make_sc_call(N, C, D, H, W):
    PW = N // NW
    NB = PW // B
    NG = B // L

    mesh = plsc.VectorSubcoreMesh(core_axis_name="c", subcore_axis_name="s")

    @functools.partial(
        pl.kernel,
        mesh=mesh,
        compiler_params=pltpu.CompilerParams(
            needs_layout_passes=False, use_tc_tiling_on_sc=False),
        out_type=jax.ShapeDtypeStruct((N * C,), jnp.float32),
        scratch_types=[
            pltpu.VMEM((6 * L,), jnp.float32),
            pltpu.VMEM((B,), jnp.float32),
            pltpu.VMEM((B,), jnp.float32),
            pltpu.VMEM((B,), jnp.float32),
            pltpu.VMEM((NC, B), jnp.int32),
            pltpu.VMEM((NC * B,), jnp.float32),
            pltpu.VMEM((NC * B, L), jnp.float32),
            pltpu.VMEM((B * C,), jnp.float32),
            pltpu.SemaphoreType.DMA,
        ],
    )
    def launch(xs_hbm, ys_hbm, zs_hbm, table_hbm, params_hbm, out_hbm,
               par_v, x_v, y_v, z_v, idx_v, w_v, rows_v, out_v, sem):
        wid = lax.axis_index("s") * 2 + lax.axis_index("c")
        base = wid * PW
        pltpu.sync_copy(params_hbm, par_v)
        iota = lax.iota(jnp.int32, L)
        zero = iota - iota
        mns = [par_v[pl.ds(a * L, L)] for a in range(3)]
        dns = [par_v[pl.ds((3 + a) * L, L)] for a in range(3)]

        def axis_vals(p, a, S):
            n = (p - mns[a]) / dns[a]
            cc = n * 2.0 - 1.0
            u = (cc + 1.0) * 0.5 * float(S - 1)
            u = jnp.clip(u, 0.0, float(S - 1))
            i0 = u.astype(jnp.int32)
            t = u - i0.astype(jnp.float32)
            i1 = jnp.minimum(i0 + 1, S - 1)
            return i0, i1, t

        def block_body(bi, carry):
            pbase = base + bi * B
            pltpu.sync_copy(xs_hbm.at[pl.ds(pbase, B)], x_v)
            pltpu.sync_copy(ys_hbm.at[pl.ds(pbase, B)], y_v)
            pltpu.sync_copy(zs_hbm.at[pl.ds(pbase, B)], z_v)

            for g in range(NG):
                off = g * L
                sl = pl.ds(off, L)
                iz0, iz1, tz = axis_vals(x_v[sl], 0, D)
                iy0, iy1, ty = axis_vals(y_v[sl], 1, H)
                ix0, ix1, tx = axis_vals(z_v[sl], 2, W)
                a0 = iz0 * (H * W)
                a1 = iz1 * (H * W)
                b0 = iy0 * W
                b1 = iy1 * W
                r00 = a0 + b0
                r01 = a0 + b1
                r10 = a1 + b0
                r11 = a1 + b1
                idx_v[0, sl] = r00 + ix0
                idx_v[1, sl] = r00 + ix1
                idx_v[2, sl] = r01 + ix0
                idx_v[3, sl] = r01 + ix1
                idx_v[4, sl] = r10 + ix0
                idx_v[5, sl] = r10 + ix1
                idx_v[6, sl] = r11 + ix0
                idx_v[7, sl] = r11 + ix1
                cz = 1.0 - tz
                cy = 1.0 - ty
                cx = 1.0 - tx
                w00 = cz * cy
                w01 = cz * ty
                w10 = tz * cy
                w11 = tz * ty
                w_v[pl.ds(0 * B + off, L)] = w00 * cx
                w_v[pl.ds(1 * B + off, L)] = w00 * tx
                w_v[pl.ds(2 * B + off, L)] = w01 * cx
                w_v[pl.ds(3 * B + off, L)] = w01 * tx
                w_v[pl.ds(4 * B + off, L)] = w10 * cx
                w_v[pl.ds(5 * B + off, L)] = w10 * tx
                w_v[pl.ds(6 * B + off, L)] = w11 * cx
                w_v[pl.ds(7 * B + off, L)] = w11 * tx

            cps = [pltpu.async_copy(table_hbm.at[idx_v.at[k]],
                                    rows_v.at[pl.ds(k * B, B), :], sem)
                   for k in range(NC)]
            for cp in cps:
                cp.wait()

            for g in range(NG):
                off = g * L
                pv = iota + off
                wks = [w_v[pl.ds(k * B + off, L)] for k in range(NC)]
                ov = pv * C
                for ch in range(C):
                    chv = zero + ch
                    acc = wks[0] * plsc.load_gather(rows_v, [pv, chv])
                    for k in range(1, NC):
                        rv = pv + (k * B)
                        acc = acc + wks[k] * plsc.load_gather(rows_v, [rv, chv])
                    plsc.store_scatter(out_v, [ov + ch], acc)

            pltpu.sync_copy(out_v, out_hbm.at[pl.ds(pbase * C, B * C)])
            return carry

        lax.fori_loop(0, NB, block_body, 0)

    return launch


def kernel(xyz, grid, xyz_min, xyz_max):
    N = xyz.shape[0]
    C = grid.shape[1]
    D, H, W = grid.shape[2], grid.shape[3], grid.shape[4]
    V = D * H * W
    table = jnp.pad(grid[0].reshape(C, V).T, ((0, 0), (0, L - C)))
    xs = xyz[:, 0]
    ys = xyz[:, 1]
    zs = xyz[:, 2]
    den = xyz_max - xyz_min
    params = jnp.concatenate([
        jnp.broadcast_to(xyz_min[:, None], (3, L)).reshape(-1),
        jnp.broadcast_to(den[:, None], (3, L)).reshape(-1),
    ])
    call = _make_sc_call(N, C, D, H, W)
    return call(xs, ys, zs, table, params).reshape(N, C)


# speedup vs baseline: 2.0949x; 1.6010x over previous
import functools

import jax
import jax.numpy as jnp
from jax import lax
from jax.experimental import pallas as pl
from jax.experimental.pallas import tpu as pltpu
from jax.experimental.pallas import tpu_sc as plsc

L = 16
NW = 32
B = 128
NC = 8
T = 2000

_PARAMS = pltpu.CompilerParams(
    needs_layout_passes=False, use_tc_tiling_on_sc=False)


def _make_format_call(C, V):
    VW = V // NW
    NB = VW // T
    NJ = T // L

    mesh = plsc.VectorSubcoreMesh(core_axis_name="c", subcore_axis_name="s")

    @functools.partial(
        pl.kernel,
        mesh=mesh,
        compiler_params=_PARAMS,
        out_type=jax.ShapeDtypeStruct((V * L,), jnp.float32),
        scratch_types=[
            pltpu.VMEM((C, T), jnp.float32),
            pltpu.VMEM((T * L,), jnp.float32),
            pltpu.SemaphoreType.DMA,
        ],
    )
    def fmt(grid_hbm, table_hbm, ch_v, out_v, sem):
        wid = lax.axis_index("s") * 2 + lax.axis_index("c")
        v0 = wid * VW
        iota = lax.iota(jnp.int32, L)
        zero = iota - iota
        zf = jnp.zeros((L,), jnp.float32)

        def zf_body(j, carry):
            out_v[pl.ds(j * L, L)] = zf
            return carry

        lax.fori_loop(0, NJ * L, zf_body, 0)

        bases = [iota * L + c for c in range(C)]

        def block_body(bi, carry):
            vb = v0 + bi * T
            cps = [pltpu.async_copy(
                       grid_hbm.at[pl.ds(c * V + vb, T)], ch_v.at[c], sem)
                   for c in range(C)]
            for cp in cps:
                cp.wait()
            for j in range(NJ):
                jof = j * (L * L)
                sl = pl.ds(j * L, L)
                for c in range(C):
                    plsc.store_scatter(out_v, [bases[c] + jof], ch_v[c, sl])
            pltpu.sync_copy(out_v, table_hbm.at[pl.ds(vb * L, T * L)])
            return carry

        lax.fori_loop(0, NB, block_body, 0)

    return fmt


def _make_sample_call(N, C, D, H, W):
    PW = N // NW
    NB = PW // B
    NG = B // L

    mesh = plsc.VectorSubcoreMesh(core_axis_name="c", subcore_axis_name="s")

    @functools.partial(
        pl.kernel,
        mesh=mesh,
        compiler_params=_PARAMS,
        out_type=jax.ShapeDtypeStruct((N * C,), jnp.float32),
        scratch_types=[
            pltpu.VMEM((6 * L,), jnp.float32),
            pltpu.VMEM((3 * B,), jnp.float32),
            pltpu.VMEM((NC, B), jnp.int32),
            pltpu.VMEM((NC * B,), jnp.float32),
            pltpu.VMEM((NC * B, L), jnp.float32),
            pltpu.VMEM((B * C,), jnp.float32),
            pltpu.SemaphoreType.DMA,
        ],
    )
    def launch(xyz_hbm, table_hbm, params_hbm, out_hbm,
               par_v, xyz_v, idx_v, w_v, rows_v, out_v, sem):
        wid = lax.axis_index("s") * 2 + lax.axis_index("c")
        base = wid * PW
        pltpu.sync_copy(params_hbm, par_v)
        iota = lax.iota(jnp.int32, L)
        zero = iota - iota
        iota3 = iota * 3
        mns = [par_v[pl.ds(a * L, L)] for a in range(3)]
        dns = [par_v[pl.ds((3 + a) * L, L)] for a in range(3)]

        def axis_vals(p, a, S):
            n = (p - mns[a]) / dns[a]
            cc = n * 2.0 - 1.0
            u = (cc + 1.0) * 0.5 * float(S - 1)
            u = jnp.clip(u, 0.0, float(S - 1))
            i0 = u.astype(jnp.int32)
            t = u - i0.astype(jnp.float32)
            i1 = jnp.minimum(i0 + 1, S - 1)
            return i0, i1, t

        def block_body(bi, carry):
            pbase = base + bi * B
            pltpu.sync_copy(xyz_hbm.at[pl.ds(pbase * 3, 3 * B)], xyz_v)

            for g in range(NG):
                off = g * L
                px = plsc.load_gather(xyz_v, [iota3 + (3 * off + 0)])
                py = plsc.load_gather(xyz_v, [iota3 + (3 * off + 1)])
                pz = plsc.load_gather(xyz_v, [iota3 + (3 * off + 2)])
                sl = pl.ds(off, L)
                iz0, iz1, tz = axis_vals(px, 0, D)
                iy0, iy1, ty = axis_vals(py, 1, H)
                ix0, ix1, tx = axis_vals(pz, 2, W)
                a0 = iz0 * (H * W)
                a1 = iz1 * (H * W)
                b0 = iy0 * W
                b1 = iy1 * W
                r00 = a0 + b0
                r01 = a0 + b1
                r10 = a1 + b0
                r11 = a1 + b1
                idx_v[0, sl] = r00 + ix0
                idx_v[1, sl] = r00 + ix1
                idx_v[2, sl] = r01 + ix0
                idx_v[3, sl] = r01 + ix1
                idx_v[4, sl] = r10 + ix0
                idx_v[5, sl] = r10 + ix1
                idx_v[6, sl] = r11 + ix0
                idx_v[7, sl] = r11 + ix1
                cz = 1.0 - tz
                cy = 1.0 - ty
                cx = 1.0 - tx
                w00 = cz * cy
                w01 = cz * ty
                w10 = tz * cy
                w11 = tz * ty
                w_v[pl.ds(0 * B + off, L)] = w00 * cx
                w_v[pl.ds(1 * B + off, L)] = w00 * tx
                w_v[pl.ds(2 * B + off, L)] = w01 * cx
                w_v[pl.ds(3 * B + off, L)] = w01 * tx
                w_v[pl.ds(4 * B + off, L)] = w10 * cx
                w_v[pl.ds(5 * B + off, L)] = w10 * tx
                w_v[pl.ds(6 * B + off, L)] = w11 * cx
                w_v[pl.ds(7 * B + off, L)] = w11 * tx

            cps = [pltpu.async_copy(table_hbm.at[idx_v.at[k]],
                                    rows_v.at[pl.ds(k * B, B), :], sem)
                   for k in range(NC)]
            for cp in cps:
                cp.wait()

            for g in range(NG):
                off = g * L
                pv = iota + off
                wks = [w_v[pl.ds(k * B + off, L)] for k in range(NC)]
                ov = pv * C
                for ch in range(C):
                    chv = zero + ch
                    acc = wks[0] * plsc.load_gather(rows_v, [pv, chv])
                    for k in range(1, NC):
                        rv = pv + (k * B)
                        acc = acc + wks[k] * plsc.load_gather(rows_v, [rv, chv])
                    plsc.store_scatter(out_v, [ov + ch], acc)

            pltpu.sync_copy(out_v, out_hbm.at[pl.ds(pbase * C, B * C)])
            return carry

        lax.fori_loop(0, NB, block_body, 0)

    return launch


def kernel(xyz, grid, xyz_min, xyz_max):
    N = xyz.shape[0]
    C = grid.shape[1]
    D, H, W = grid.shape[2], grid.shape[3], grid.shape[4]
    V = D * H * W
    grid_flat = grid.reshape(C * V)
    table = _make_format_call(C, V)(grid_flat).reshape(V, L)
    xyz_flat = xyz.reshape(N * 3)
    den = xyz_max - xyz_min
    params = jnp.concatenate([
        jnp.broadcast_to(xyz_min[:, None], (3, L)).reshape(-1),
        jnp.broadcast_to(den[:, None], (3, L)).reshape(-1),
    ])
    call = _make_sample_call(N, C, D, H, W)
    return call(xyz_flat, table, params).reshape(N, C)


# speedup vs baseline: 2.1277x; 1.0157x over previous
import functools

import jax
import jax.numpy as jnp
from jax import lax
from jax.experimental import pallas as pl
from jax.experimental.pallas import tpu as pltpu
from jax.experimental.pallas import tpu_sc as plsc

L = 16
NW = 32
B = 128
NC = 8
T = 2000

_PARAMS = pltpu.CompilerParams(
    needs_layout_passes=False, use_tc_tiling_on_sc=False)


def _make_format_call(C, V):
    VW = V // NW
    NB = VW // T
    NJ = T // L

    mesh = plsc.VectorSubcoreMesh(core_axis_name="c", subcore_axis_name="s")

    @functools.partial(
        pl.kernel,
        mesh=mesh,
        compiler_params=_PARAMS,
        out_type=jax.ShapeDtypeStruct((V, L), jnp.float32),
        scratch_types=[
            pltpu.VMEM((C, T), jnp.float32),
            pltpu.VMEM((T, L), jnp.float32),
            pltpu.SemaphoreType.DMA,
        ],
    )
    def fmt(grid_hbm, table_hbm, ch_v, out_v, sem):
        wid = lax.axis_index("s") * 2 + lax.axis_index("c")
        v0 = wid * VW
        iota = lax.iota(jnp.int32, L)
        zero = iota - iota
        zf = jnp.zeros((L,), jnp.float32)

        def zf_body(j, carry):
            out_v[j, :] = zf
            return carry

        lax.fori_loop(0, T, zf_body, 0)

        def block_body(bi, carry):
            vb = v0 + bi * T
            cps = [pltpu.async_copy(
                       grid_hbm.at[pl.ds(c * V + vb, T)], ch_v.at[c], sem)
                   for c in range(C)]
            for cp in cps:
                cp.wait()
            for j in range(NJ):
                vv = iota + j * L
                sl = pl.ds(j * L, L)
                for c in range(C):
                    plsc.store_scatter(out_v, [vv, zero + c], ch_v[c, sl])
            pltpu.sync_copy(out_v, table_hbm.at[pl.ds(vb, T), :])
            return carry

        lax.fori_loop(0, NB, block_body, 0)

    return fmt


def _make_sample_call(N, C, D, H, W):
    PW = N // NW
    NB = PW // B
    NG = B // L

    mesh = plsc.VectorSubcoreMesh(core_axis_name="c", subcore_axis_name="s")

    @functools.partial(
        pl.kernel,
        mesh=mesh,
        compiler_params=_PARAMS,
        out_type=jax.ShapeDtypeStruct((N, C), jnp.float32),
        scratch_types=[
            pltpu.VMEM((6 * L,), jnp.float32),
            pltpu.VMEM((B, 3), jnp.float32),
            pltpu.VMEM((NC, B), jnp.int32),
            pltpu.VMEM((NC * B,), jnp.float32),
            pltpu.VMEM((NC * B, L), jnp.float32),
            pltpu.VMEM((B, C), jnp.float32),
            pltpu.SemaphoreType.DMA,
        ],
    )
    def launch(xyz_hbm, table_hbm, params_hbm, out_hbm,
               par_v, xyz_v, idx_v, w_v, rows_v, out_v, sem):
        wid = lax.axis_index("s") * 2 + lax.axis_index("c")
        base = wid * PW
        pltpu.sync_copy(params_hbm, par_v)
        iota = lax.iota(jnp.int32, L)
        zero = iota - iota
        mns = [par_v[pl.ds(a * L, L)] for a in range(3)]
        dns = [par_v[pl.ds((3 + a) * L, L)] for a in range(3)]

        def axis_vals(p, a, S):
            n = (p - mns[a]) / dns[a]
            cc = n * 2.0 - 1.0
            u = (cc + 1.0) * 0.5 * float(S - 1)
            u = jnp.clip(u, 0.0, float(S - 1))
            i0 = u.astype(jnp.int32)
            t = u - i0.astype(jnp.float32)
            i1 = jnp.minimum(i0 + 1, S - 1)
            return i0, i1, t

        def block_body(bi, carry):
            pbase = base + bi * B
            pltpu.sync_copy(xyz_hbm.at[pl.ds(pbase, B), :], xyz_v)

            for g in range(NG):
                off = g * L
                pg = iota + off
                px = plsc.load_gather(xyz_v, [pg, zero])
                py = plsc.load_gather(xyz_v, [pg, zero + 1])
                pz = plsc.load_gather(xyz_v, [pg, zero + 2])
                sl = pl.ds(off, L)
                iz0, iz1, tz = axis_vals(px, 0, D)
                iy0, iy1, ty = axis_vals(py, 1, H)
                ix0, ix1, tx = axis_vals(pz, 2, W)
                a0 = iz0 * (H * W)
                a1 = iz1 * (H * W)
                b0 = iy0 * W
                b1 = iy1 * W
                r00 = a0 + b0
                r01 = a0 + b1
                r10 = a1 + b0
                r11 = a1 + b1
                idx_v[0, sl] = r00 + ix0
                idx_v[1, sl] = r00 + ix1
                idx_v[2, sl] = r01 + ix0
                idx_v[3, sl] = r01 + ix1
                idx_v[4, sl] = r10 + ix0
                idx_v[5, sl] = r10 + ix1
                idx_v[6, sl] = r11 + ix0
                idx_v[7, sl] = r11 + ix1
                cz = 1.0 - tz
                cy = 1.0 - ty
                cx = 1.0 - tx
                w00 = cz * cy
                w01 = cz * ty
                w10 = tz * cy
                w11 = tz * ty
                w_v[pl.ds(0 * B + off, L)] = w00 * cx
                w_v[pl.ds(1 * B + off, L)] = w00 * tx
                w_v[pl.ds(2 * B + off, L)] = w01 * cx
                w_v[pl.ds(3 * B + off, L)] = w01 * tx
                w_v[pl.ds(4 * B + off, L)] = w10 * cx
                w_v[pl.ds(5 * B + off, L)] = w10 * tx
                w_v[pl.ds(6 * B + off, L)] = w11 * cx
                w_v[pl.ds(7 * B + off, L)] = w11 * tx

            cps = [pltpu.async_copy(table_hbm.at[idx_v.at[k]],
                                    rows_v.at[pl.ds(k * B, B), :], sem)
                   for k in range(NC)]
            for cp in cps:
                cp.wait()

            for g in range(NG):
                off = g * L
                pv = iota + off
                wks = [w_v[pl.ds(k * B + off, L)] for k in range(NC)]
                for ch in range(C):
                    chv = zero + ch
                    acc = wks[0] * plsc.load_gather(rows_v, [pv, chv])
                    for k in range(1, NC):
                        rv = pv + (k * B)
                        acc = acc + wks[k] * plsc.load_gather(rows_v, [rv, chv])
                    plsc.store_scatter(out_v, [pv, chv], acc)

            pltpu.sync_copy(out_v, out_hbm.at[pl.ds(pbase, B), :])
            return carry

        lax.fori_loop(0, NB, block_body, 0)

    return launch


def kernel(xyz, grid, xyz_min, xyz_max):
    N = xyz.shape[0]
    C = grid.shape[1]
    D, H, W = grid.shape[2], grid.shape[3], grid.shape[4]
    V = D * H * W
    grid_flat = grid.reshape(C * V)
    table = _make_format_call(C, V)(grid_flat)
    den = xyz_max - xyz_min
    params = jnp.concatenate([
        jnp.broadcast_to(xyz_min[:, None], (3, L)).reshape(-1),
        jnp.broadcast_to(den[:, None], (3, L)).reshape(-1),
    ])
    call = _make_sample_call(N, C, D, H, W)
    return call(xyz, table, params)


# speedup vs baseline: 2.3066x; 1.0841x over previous
import functools

import jax
import jax.numpy as jnp
from jax import lax
from jax.experimental import pallas as pl
from jax.experimental.pallas import tpu as pltpu
from jax.experimental.pallas import tpu_sc as plsc

L = 16
NW = 32
B = 128
NC = 8
T = 2000

_PARAMS = pltpu.CompilerParams(
    needs_layout_passes=False, use_tc_tiling_on_sc=False)


def _make_format_call(C, V):
    VW = V // NW
    NB = VW // T
    NJ = T // L

    mesh = plsc.VectorSubcoreMesh(core_axis_name="c", subcore_axis_name="s")

    @functools.partial(
        pl.kernel,
        mesh=mesh,
        compiler_params=_PARAMS,
        out_type=jax.ShapeDtypeStruct((V, L), jnp.float32),
        scratch_types=[
            pltpu.VMEM((C, T), jnp.float32),
            pltpu.VMEM((T, L), jnp.float32),
            pltpu.SemaphoreType.DMA,
        ],
    )
    def fmt(grid_hbm, table_hbm, ch_v, out_v, sem):
        wid = lax.axis_index("s") * 2 + lax.axis_index("c")
        v0 = wid * VW
        iota = lax.iota(jnp.int32, L)
        zero = iota - iota
        zf = jnp.zeros((L,), jnp.float32)

        def zf_body(j, carry):
            out_v[j, :] = zf
            return carry

        lax.fori_loop(0, T, zf_body, 0)

        def block_body(bi, carry):
            vb = v0 + bi * T
            cps = [pltpu.async_copy(
                       grid_hbm.at[pl.ds(c * V + vb, T)], ch_v.at[c], sem)
                   for c in range(C)]
            for cp in cps:
                cp.wait()
            for j in range(NJ):
                vv = iota + j * L
                sl = pl.ds(j * L, L)
                for c in range(C):
                    plsc.store_scatter(out_v, [vv, zero + c], ch_v[c, sl])
            pltpu.sync_copy(out_v, table_hbm.at[pl.ds(vb, T), :])
            return carry

        lax.fori_loop(0, NB, block_body, 0)

    return fmt


def _make_sample_call(N, C, D, H, W):
    PW = N // NW
    NB = PW // B
    NG = B // L

    mesh = plsc.VectorSubcoreMesh(core_axis_name="c", subcore_axis_name="s")

    P = 4
    LEAD = 3

    @functools.partial(
        pl.kernel,
        mesh=mesh,
        compiler_params=_PARAMS,
        out_type=jax.ShapeDtypeStruct((N, C), jnp.float32),
        scratch_types=[
            pltpu.VMEM((6 * L,), jnp.float32),
            pltpu.VMEM((P, B, 3), jnp.float32),
            pltpu.VMEM((P, NC, B), jnp.int32),
            pltpu.VMEM((P, NC, B), jnp.float32),
            pltpu.VMEM((P, NC * B, L), jnp.float32),
            pltpu.VMEM((P, B, C), jnp.float32),
            pltpu.SemaphoreType.DMA((P,)),
            pltpu.SemaphoreType.DMA((P,)),
        ],
    )
    def launch(xyz_hbm, table_hbm, params_hbm, out_hbm,
               par_v, xyz_v, idx_v, w_v, rows_v, out_v, gsem, osem):
        wid = lax.axis_index("s") * 2 + lax.axis_index("c")
        base = wid * PW
        pltpu.sync_copy(params_hbm, par_v)
        iota = lax.iota(jnp.int32, L)
        zero = iota - iota
        mns = [par_v[pl.ds(a * L, L)] for a in range(3)]
        dns = [par_v[pl.ds((3 + a) * L, L)] for a in range(3)]

        def axis_vals(p, a, S):
            n = (p - mns[a]) / dns[a]
            cc = n * 2.0 - 1.0
            u = (cc + 1.0) * 0.5 * float(S - 1)
            u = jnp.clip(u, 0.0, float(S - 1))
            i0 = u.astype(jnp.int32)
            t = u - i0.astype(jnp.float32)
            i1 = jnp.minimum(i0 + 1, S - 1)
            return i0, i1, t

        def fire(b, p):
            pbase = base + b * B
            pltpu.sync_copy(xyz_hbm.at[pl.ds(pbase, B), :], xyz_v.at[p])
            pfv = zero + p
            for g in range(NG):
                off = g * L
                pg = iota + off
                px = plsc.load_gather(xyz_v, [pfv, pg, zero])
                py = plsc.load_gather(xyz_v, [pfv, pg, zero + 1])
                pz = plsc.load_gather(xyz_v, [pfv, pg, zero + 2])
                sl = pl.ds(off, L)
                iz0, iz1, tz = axis_vals(px, 0, D)
                iy0, iy1, ty = axis_vals(py, 1, H)
                ix0, ix1, tx = axis_vals(pz, 2, W)
                a0 = iz0 * (H * W)
                a1 = iz1 * (H * W)
                b0 = iy0 * W
                b1 = iy1 * W
                r00 = a0 + b0
                r01 = a0 + b1
                r10 = a1 + b0
                r11 = a1 + b1
                idx_v[p, 0, sl] = r00 + ix0
                idx_v[p, 1, sl] = r00 + ix1
                idx_v[p, 2, sl] = r01 + ix0
                idx_v[p, 3, sl] = r01 + ix1
                idx_v[p, 4, sl] = r10 + ix0
                idx_v[p, 5, sl] = r10 + ix1
                idx_v[p, 6, sl] = r11 + ix0
                idx_v[p, 7, sl] = r11 + ix1
                cz = 1.0 - tz
                cy = 1.0 - ty
                cx = 1.0 - tx
                w00 = cz * cy
                w01 = cz * ty
                w10 = tz * cy
                w11 = tz * ty
                w_v[p, 0, sl] = w00 * cx
                w_v[p, 1, sl] = w00 * tx
                w_v[p, 2, sl] = w01 * cx
                w_v[p, 3, sl] = w01 * tx
                w_v[p, 4, sl] = w10 * cx
                w_v[p, 5, sl] = w10 * tx
                w_v[p, 6, sl] = w11 * cx
                w_v[p, 7, sl] = w11 * tx
            for k in range(NC):
                pltpu.async_copy(table_hbm.at[idx_v.at[p, k]],
                                 rows_v.at[p, pl.ds(k * B, B), :],
                                 gsem.at[p])

        def gwait(p):
            for k in range(NC):
                pltpu.make_async_copy(table_hbm.at[idx_v.at[p, k]],
                                      rows_v.at[p, pl.ds(k * B, B), :],
                                      gsem.at[p]).wait()

        def owait(b, p):
            pbase = base + b * B
            pltpu.make_async_copy(out_v.at[p],
                                  out_hbm.at[pl.ds(pbase, B), :],
                                  osem.at[p]).wait()

        def finish(b, p):
            pbase = base + b * B
            gwait(p)
            pfv = zero + p
            for g in range(NG):
                off = g * L
                pv = iota + off
                sl = pl.ds(off, L)
                wks = [w_v[p, k, sl] for k in range(NC)]
                for ch in range(C):
                    chv = zero + ch
                    acc = wks[0] * plsc.load_gather(rows_v, [pfv, pv, chv])
                    for k in range(1, NC):
                        rv = pv + (k * B)
                        acc = acc + wks[k] * plsc.load_gather(
                            rows_v, [pfv, rv, chv])
                    plsc.store_scatter(out_v, [pfv, pv, chv], acc)
            pltpu.async_copy(out_v.at[p],
                             out_hbm.at[pl.ds(pbase, B), :],
                             osem.at[p])

        for j in range(LEAD):
            fire(j, j)

        def block_body(t, carry):
            p = lax.bitwise_and(t, P - 1)
            pn = lax.bitwise_and(t + LEAD, P - 1)

            @pl.when(t + LEAD < NB)
            def _():
                fire(t + LEAD, pn)

            @pl.when(t >= P)
            def _():
                owait(t - P, p)

            finish(t, p)
            return carry

        lax.fori_loop(0, NB, block_body, 0)

        for j in range(NB - P, NB):
            owait(j, j % P)

    return launch


def kernel(xyz, grid, xyz_min, xyz_max):
    N = xyz.shape[0]
    C = grid.shape[1]
    D, H, W = grid.shape[2], grid.shape[3], grid.shape[4]
    V = D * H * W
    grid_flat = grid.reshape(C * V)
    table = _make_format_call(C, V)(grid_flat)
    den = xyz_max - xyz_min
    params = jnp.concatenate([
        jnp.broadcast_to(xyz_min[:, None], (3, L)).reshape(-1),
        jnp.broadcast_to(den[:, None], (3, L)).reshape(-1),
    ])
    call = _make_sample_call(N, C, D, H, W)
    return call(xyz, table, params)
